# baseline (speedup 1.0000x reference)
import math

import numpy as np
import jax
import jax.numpy as jnp
from jax.experimental import pallas as pl
from jax.experimental.pallas import tpu as pltpu

_N_STEPS = 200
_PAD = 256
_BATCH = 1024
_FEAT = 512 * 4 * 4
_BLK_B = 128


def _make_tables():
    steps = np.arange(_N_STEPS + 1, dtype=np.float64)
    tt = steps / _N_STEPS
    ac = np.cos((tt + 0.008) / 1.008 * math.pi / 2.0) ** 2
    ac = ac / ac[0]
    betas = np.clip(1.0 - ac[1:] / ac[:-1], 0.0001, 0.9999).astype(np.float32)
    alphas = (1.0 - betas).astype(np.float32)
    acp = np.cumprod(alphas, axis=0)
    sa = np.sqrt(acp).astype(np.float32)
    so = np.sqrt(1.0 - acp).astype(np.float32)
    pa = np.zeros((1, _PAD), np.float32)
    po = np.zeros((1, _PAD), np.float32)
    pa[0, :_N_STEPS] = sa
    po[0, :_N_STEPS] = so
    return pa, po


_TBL_AC, _TBL_OM = _make_tables()


def _body(t_ref, ta_ref, to_ref, x_ref, n_ref, o_ref):
    g = pl.program_id(0)
    t_col = t_ref[pl.ds(g * _BLK_B, _BLK_B), :]
    k = jax.lax.broadcasted_iota(jnp.int32, (_BLK_B, _PAD), 1)
    onehot = t_col == k
    a = jnp.sum(jnp.where(onehot, ta_ref[:, :], 0.0), axis=1, keepdims=True)
    b = jnp.sum(jnp.where(onehot, to_ref[:, :], 0.0), axis=1, keepdims=True)
    a4 = a.reshape(_BLK_B, 1, 1, 1)
    b4 = b.reshape(_BLK_B, 1, 1, 1)
    o_ref[...] = a4 * x_ref[...] + b4 * n_ref[...]


def kernel(x0, t, noise):
    xv = jnp.transpose(x0, (0, 2, 3, 1))
    nv = jnp.transpose(noise, (0, 2, 3, 1))
    t2 = t.astype(jnp.int32).reshape(_BATCH, 1)
    grid = (_BATCH // _BLK_B, 4)
    out = pl.pallas_call(
        _body,
        grid=grid,
        in_specs=[
            pl.BlockSpec((_BATCH, 1), lambda g, h: (0, 0)),
            pl.BlockSpec((1, _PAD), lambda g, h: (0, 0)),
            pl.BlockSpec((1, _PAD), lambda g, h: (0, 0)),
            pl.BlockSpec((_BLK_B, 1, 4, 512), lambda g, h: (g, h, 0, 0)),
            pl.BlockSpec((_BLK_B, 1, 4, 512), lambda g, h: (g, h, 0, 0)),
        ],
        out_specs=pl.BlockSpec((_BLK_B, 1, 4, 512), lambda g, h: (g, h, 0, 0)),
        out_shape=jax.ShapeDtypeStruct((_BATCH, 4, 4, 512), jnp.float32),
    )(t2, jnp.asarray(_TBL_AC), jnp.asarray(_TBL_OM), xv, nv)
    return jnp.transpose(out, (0, 3, 1, 2))


# speedup vs baseline: 1.2612x; 1.2612x over previous
import math

import numpy as np
import jax
import jax.numpy as jnp
from jax.experimental import pallas as pl
from jax.experimental.pallas import tpu as pltpu

_N_STEPS = 200
_PAD = 256
_BATCH = 1024
_FEAT = 512 * 4 * 4
_BLK_B = 128


def _make_tables():
    steps = np.arange(_N_STEPS + 1, dtype=np.float64)
    tt = steps / _N_STEPS
    ac = np.cos((tt + 0.008) / 1.008 * math.pi / 2.0) ** 2
    ac = ac / ac[0]
    betas = np.clip(1.0 - ac[1:] / ac[:-1], 0.0001, 0.9999).astype(np.float32)
    alphas = (1.0 - betas).astype(np.float32)
    acp = np.cumprod(alphas, axis=0)
    sa = np.sqrt(acp).astype(np.float32)
    so = np.sqrt(1.0 - acp).astype(np.float32)
    pa = np.zeros((1, _PAD), np.float32)
    po = np.zeros((1, _PAD), np.float32)
    pa[0, :_N_STEPS] = sa
    po[0, :_N_STEPS] = so
    return pa, po


_TBL_AC, _TBL_OM = _make_tables()


def _body(t_ref, ta_ref, to_ref, x_ref, n_ref, o_ref):
    g = pl.program_id(0)
    t_col = t_ref[pl.ds(g * _BLK_B, _BLK_B), :]
    k = jax.lax.broadcasted_iota(jnp.int32, (_BLK_B, _PAD), 1)
    onehot = t_col == k
    a = jnp.sum(jnp.where(onehot, ta_ref[:, :], 0.0), axis=1, keepdims=True)
    b = jnp.sum(jnp.where(onehot, to_ref[:, :], 0.0), axis=1, keepdims=True)
    a4 = a.reshape(_BLK_B, 1, 1, 1)
    b4 = b.reshape(_BLK_B, 1, 1, 1)
    o_ref[...] = a4 * x_ref[...] + b4 * n_ref[...]


def kernel(x0, t, noise):
    xv = jnp.transpose(x0, (0, 2, 3, 1))
    nv = jnp.transpose(noise, (0, 2, 3, 1))
    t2 = t.astype(jnp.int32).reshape(_BATCH, 1)
    grid = (_BATCH // _BLK_B,)
    out = pl.pallas_call(
        _body,
        grid=grid,
        in_specs=[
            pl.BlockSpec((_BATCH, 1), lambda g: (0, 0)),
            pl.BlockSpec((1, _PAD), lambda g: (0, 0)),
            pl.BlockSpec((1, _PAD), lambda g: (0, 0)),
            pl.BlockSpec((_BLK_B, 4, 4, 512), lambda g: (g, 0, 0, 0)),
            pl.BlockSpec((_BLK_B, 4, 4, 512), lambda g: (g, 0, 0, 0)),
        ],
        out_specs=pl.BlockSpec((_BLK_B, 4, 4, 512), lambda g: (g, 0, 0, 0)),
        out_shape=jax.ShapeDtypeStruct((_BATCH, 4, 4, 512), jnp.float32),
    )(t2, jnp.asarray(_TBL_AC), jnp.asarray(_TBL_OM), xv, nv)
    return jnp.transpose(out, (0, 3, 1, 2))


# speedup vs baseline: 1.3591x; 1.0776x over previous
import math

import numpy as np
import jax
import jax.numpy as jnp
from jax.experimental import pallas as pl
from jax.experimental.pallas import tpu as pltpu

_N_STEPS = 200
_PAD = 256
_BATCH = 1024
_FEAT = 512 * 4 * 4
_BLK_B = 128


def _make_tables():
    steps = np.arange(_N_STEPS + 1, dtype=np.float64)
    tt = steps / _N_STEPS
    ac = np.cos((tt + 0.008) / 1.008 * math.pi / 2.0) ** 2
    ac = ac / ac[0]
    betas = np.clip(1.0 - ac[1:] / ac[:-1], 0.0001, 0.9999).astype(np.float32)
    alphas = (1.0 - betas).astype(np.float32)
    acp = np.cumprod(alphas, axis=0)
    sa = np.sqrt(acp).astype(np.float32)
    so = np.sqrt(1.0 - acp).astype(np.float32)
    pa = np.zeros((1, _PAD), np.float32)
    po = np.zeros((1, _PAD), np.float32)
    pa[0, :_N_STEPS] = sa
    po[0, :_N_STEPS] = so
    return pa, po


_TBL_AC, _TBL_OM = _make_tables()


def _body(t_ref, ta_ref, to_ref, x_ref, n_ref, o_ref):
    g = pl.program_id(0)
    t_row = t_ref[pl.ds(g, 1), :]
    t_col = jnp.transpose(t_row, (1, 0))
    k = jax.lax.broadcasted_iota(jnp.int32, (_BLK_B, _PAD), 1)
    onehot = t_col == k
    a = jnp.sum(jnp.where(onehot, ta_ref[:, :], 0.0), axis=1, keepdims=True)
    b = jnp.sum(jnp.where(onehot, to_ref[:, :], 0.0), axis=1, keepdims=True)
    a4 = a.reshape(_BLK_B, 1, 1, 1)
    b4 = b.reshape(_BLK_B, 1, 1, 1)
    o_ref[...] = a4 * x_ref[...] + b4 * n_ref[...]


def kernel(x0, t, noise):
    xv = jnp.transpose(x0, (0, 2, 3, 1))
    nv = jnp.transpose(noise, (0, 2, 3, 1))
    t2 = t.astype(jnp.int32).reshape(_BATCH // _BLK_B, _BLK_B)
    grid = (_BATCH // _BLK_B,)
    out = pl.pallas_call(
        _body,
        grid=grid,
        in_specs=[
            pl.BlockSpec((_BATCH // _BLK_B, _BLK_B), lambda g: (0, 0)),
            pl.BlockSpec((1, _PAD), lambda g: (0, 0)),
            pl.BlockSpec((1, _PAD), lambda g: (0, 0)),
            pl.BlockSpec((_BLK_B, 4, 4, 512), lambda g: (g, 0, 0, 0)),
            pl.BlockSpec((_BLK_B, 4, 4, 512), lambda g: (g, 0, 0, 0)),
        ],
        out_specs=pl.BlockSpec((_BLK_B, 4, 4, 512), lambda g: (g, 0, 0, 0)),
        out_shape=jax.ShapeDtypeStruct((_BATCH, 4, 4, 512), jnp.float32),
    )(t2, jnp.asarray(_TBL_AC), jnp.asarray(_TBL_OM), xv, nv)
    return jnp.transpose(out, (0, 3, 1, 2))
